# baseline (speedup 1.0000x reference)
import functools
import jax
import jax.numpy as jnp
from jax import lax
from jax.experimental import pallas as pl
from jax.experimental.pallas import tpu as pltpu
from jax.experimental.pallas import tpu_sc as plsc

_KD = 5
_NB = 4
_NQ = 8


def _elu(v):
    return jnp.where(v > 0, v, jnp.exp(v) - 1.0)


def _bd8(m):
    eye = jnp.eye(8, dtype=jnp.float32)
    return (eye[:, None, :, None] * m.astype(jnp.float32)[None, :, None, :]).reshape(128, 128)


def _prep_body(E, V, c_ref, a0_ref, a1_ref, gb_ref, wg_ref):
    c = c_ref[...]
    a0 = a0_ref[...]
    a1 = a1_ref[...]
    v0 = a0 * (_KD - 1.0)
    b0 = jnp.floor(v0)
    f0 = v0 - b0
    i0 = b0.astype(jnp.int32)
    v1 = a1 * (_KD - 1.0)
    b1 = jnp.floor(v1)
    f1 = v1 - b1
    i1 = b1.astype(jnp.int32)
    valid = lax.broadcasted_iota(jnp.int32, c.shape, 1) < E
    for s in range(4):
        bit0 = s & 1
        bit1 = (s >> 1) & 1
        basis = (f0 if bit0 else 1.0 - f0) * (f1 if bit1 else 1.0 - f1)
        wi = jnp.clip(i0 + bit0, 0, _KD - 1) + _KD * jnp.clip(i1 + bit1, 0, _KD - 1)
        gb_ref[s : s + 1, :] = (wi * V + c) * _NB
        wg_ref[s : s + 1, :] = jnp.where(valid, basis, 0.0)


def _y_body(x_ref, w_ref, y_ref):
    y_ref[...] = jnp.dot(x_ref[...], w_ref[0], preferred_element_type=jnp.float32)


def _comb_body(x_ref, agg_ref, root_ref, cw_ref, b_ref, cb_ref, o_ref):
    xb = x_ref[...]
    agg = agg_ref[0] + agg_ref[1]
    h = agg + jnp.dot(xb, root_ref[...], preferred_element_type=jnp.float32) + b_ref[...]
    h = _elu(h)
    res = _elu(jnp.dot(xb, cw_ref[...], preferred_element_type=jnp.float32) + cb_ref[...])
    o_ref[...] = _elu(h + res)


def _make_sc_agg(V, C, EP):
    ESL = EP // _NQ
    SCH = 256
    EB = 8
    RB = 4 * EB
    BPS = SCH // EB
    NSC = ESL // SCH
    GRP = 128
    NGRP = ESL // GRP
    ROWS_SH = _NB * V
    mesh = plsc.VectorSubcoreMesh(core_axis_name="c", subcore_axis_name="s")
    NC, NS = mesh.num_cores, mesh.num_subcores
    ZT = 10
    ZR = ROWS_SH // ZT

    @functools.partial(
        pl.kernel,
        out_type=jax.ShapeDtypeStruct((NC, ROWS_SH, 8 * C), jnp.float32),
        mesh=mesh,
        scratch_types=[
            pltpu.VMEM_SHARED((ROWS_SH, 8 * C), jnp.float32),
            pltpu.VMEM((8, 128), jnp.int32),
            pltpu.VMEM((8, 128), jnp.int32),
            pltpu.VMEM((NGRP, 128), jnp.int32),
            pltpu.VMEM((4 * SCH + 16,), jnp.float32),
            pltpu.VMEM((4 * SCH + 16,), jnp.float32),
            pltpu.VMEM((2, RB, 8 * C), jnp.float32),
            pltpu.VMEM((2, GRP, 8 * C), jnp.float32),
            pltpu.SemaphoreType.DMA,
            pltpu.SemaphoreType.DMA,
            pltpu.SemaphoreType.DMA,
        ],
    )
    def sc_agg(y5, gb2, rme2, wg, out_hbm, acc_sh, gbc, gbn, rmv, wgc, wgn,
               rows, cbuf, sem_g, sem_s, sem_m):
        cid = lax.axis_index("c")
        sid = lax.axis_index("s")
        b = sid % _NB
        q = cid * (_NQ // NC) + sid // _NB
        bv = b * V

        zero16 = jnp.zeros((16,), jnp.float32)

        def zb(i, carry):
            for j8 in range(8):
                cbuf[0, i, pl.ds(j8 * 16, 16)] = zero16
            return carry

        lax.fori_loop(0, GRP, zb, 0)

        @pl.when(sid < ZT)
        def _():
            z0 = sid * ZR
            done = 0
            while done < ZR:
                n = min(GRP, ZR - done)
                pltpu.sync_copy(cbuf.at[0, pl.ds(0, n)], acc_sh.at[pl.ds(z0 + done, n)])
                done += n

        plsc.subcore_barrier()

        prow = pl.multiple_of(q * (4 * ESL // 128), 8)
        ppair = pl.multiple_of(q * 4 * ESL, 128)
        prme = pl.multiple_of(q * NGRP, 8)

        pltpu.sync_copy(rme2.at[pl.ds(prme, NGRP)], rmv)
        for j in range(NGRP):
            for l in range(8):
                sl = pl.ds(l * 16, 16)
                rmv[j, sl] = rmv[j, sl] + b

        def issue_meta(s):
            ro = pl.multiple_of(prow + s * 8, 8)
            pltpu.async_copy(gb2.at[pl.ds(ro, 8)], gbn, sem_m)
            pltpu.async_copy(wg.at[pl.ds(ppair + s * 4 * SCH, 4 * SCH)], wgn.at[pl.ds(0, 4 * SCH)], sem_m)

        def drain_meta():
            pltpu.make_async_copy(gb2.at[pl.ds(0, 8)], gbn, sem_m).wait()
            pltpu.make_async_copy(wg.at[pl.ds(0, 4 * SCH)], wgn.at[pl.ds(0, 4 * SCH)], sem_m).wait()

        def adopt_meta():
            for j in range(8):
                for l in range(8):
                    sl = pl.ds(l * 16, 16)
                    gbc[j, sl] = gbn[j, sl] + b
            def wcp(i, carry):
                sl = pl.ds(i * 16, 16)
                wgc[sl] = wgn[sl]
                return carry
            lax.fori_loop(0, (4 * SCH) // 16, wcp, 0)

        def issue_gather(u):
            idx = gbc.at[u // 4, pl.ds((u % 4) * RB, RB)]
            return pltpu.async_copy(y5.at[idx], rows.at[u % 2], sem_g)

        def drain_scatter():
            pltpu.make_async_copy(out_hbm.at[0, pl.ds(0, GRP)], cbuf.at[0], sem_s).wait()

        def combine(u):
            cb = (u // 16) % 2
            cb0 = (u % 16) * EB
            p = u % 2
            woff = u * 4 * EB
            wga = wgc[pl.ds(woff, 16)]
            wgb = wgc[pl.ds(woff + 16, 16)]
            ws = [wga[i] for i in range(16)] + [wgb[i] for i in range(16)]

            def jb(j, carry):
                sl = pl.ds(j * 16, 16)
                for e in range(EB):
                    rb = e * 4
                    v = (rows[p, rb, sl] * ws[4 * e] + rows[p, rb + 1, sl] * ws[4 * e + 1]
                         + rows[p, rb + 2, sl] * ws[4 * e + 2] + rows[p, rb + 3, sl] * ws[4 * e + 3])
                    cbuf[cb, cb0 + e, sl] = v
                return carry

            lax.fori_loop(0, 8, jb, 0)

        pltpu.sync_copy(gb2.at[pl.ds(prow, 8)], gbn)
        pltpu.sync_copy(wg.at[pl.ds(ppair, 4 * SCH)], wgn.at[pl.ds(0, 4 * SCH)])
        adopt_meta()
        issue_meta(1)

        def body(s, carry):
            @pl.when(s > 0)
            def _():
                drain_scatter()
                drain_scatter()

            pend = [issue_gather(0)]
            for u in range(BPS):
                if u + 1 < BPS:
                    pend.append(issue_gather(u + 1))
                pend[u].wait()
                combine(u)
                if u == BPS // 2 - 1:
                    pltpu.async_copy(cbuf.at[0], acc_sh.at[rmv.at[2 * s]], sem_s, add=True)
                if u == BPS - 1:
                    pltpu.async_copy(cbuf.at[1], acc_sh.at[rmv.at[2 * s + 1]], sem_s, add=True)

            @pl.when(s + 1 < NSC)
            def _():
                drain_meta()
                adopt_meta()

                @pl.when(s + 2 < NSC)
                def _():
                    issue_meta(s + 2)

            return carry

        lax.fori_loop(0, NSC, body, 0)
        drain_scatter()
        drain_scatter()
        plsc.subcore_barrier()

        @pl.when(sid < ZT)
        def _():
            pltpu.sync_copy(
                acc_sh.at[pl.ds(sid * ZR, ZR)],
                out_hbm.at[cid, pl.ds(sid * ZR, ZR)],
            )

    return sc_agg


def kernel(x, edge_index, edge_attr, W, root, bias, conv_w, conv_b):
    N, V, C, T = x.shape
    G = N * T
    E = edge_index.shape[1] // N
    K = W.shape[0]
    EP = ((E + 2047) // 2048) * 2048
    PAIRS = 4 * EP

    x = x.astype(jnp.float32)
    xt = jnp.transpose(x, (1, 3, 0, 2)).reshape(V, G, C)
    X5 = xt.reshape(V * _NB, 8 * C)

    r = edge_index[0, :E].astype(jnp.int32)
    c = edge_index[1, :E].astype(jnp.int32)
    cp_ = jnp.pad(c, (0, EP - E)).reshape(1, EP)
    a0 = jnp.pad(edge_attr[:E, 0].astype(jnp.float32), (0, EP - E)).reshape(1, EP)
    a1 = jnp.pad(edge_attr[:E, 1].astype(jnp.float32), (0, EP - E)).reshape(1, EP)

    gb, wg = pl.pallas_call(
        functools.partial(_prep_body, E, V),
        out_shape=[
            jax.ShapeDtypeStruct((4, EP), jnp.int32),
            jax.ShapeDtypeStruct((4, EP), jnp.float32),
        ],
    )(cp_, a0, a1)

    gb2 = jnp.transpose(gb).reshape(PAIRS // 128, 128)
    wgf = jnp.transpose(wg).reshape(PAIRS)
    rme2 = jnp.pad(r * 4, (0, EP - E)).reshape(EP // 128, 128)

    eye8 = jnp.eye(8, dtype=jnp.float32)
    BDW = (eye8[None, :, None, :, None]
           * W.astype(jnp.float32)[:, None, :, None, :]).reshape(K, 8 * C, 8 * C)
    Y5 = pl.pallas_call(
        _y_body,
        grid=(K,),
        in_specs=[
            pl.BlockSpec((V * _NB, 8 * C), lambda k: (0, 0)),
            pl.BlockSpec((1, 8 * C, 8 * C), lambda k: (k, 0, 0)),
        ],
        out_specs=pl.BlockSpec((V * _NB, 8 * C), lambda k: (k, 0)),
        out_shape=jax.ShapeDtypeStruct((K * V * _NB, 8 * C), jnp.float32),
    )(X5, BDW)

    agg = _make_sc_agg(V, C, EP)(Y5, gb2, rme2, wgf)

    BDroot = _bd8(root)
    BDconv = _bd8(jnp.transpose(conv_w))
    biasb = jnp.tile(bias.astype(jnp.float32), 8).reshape(1, 8 * C)
    convbb = jnp.tile(conv_b.astype(jnp.float32), 8).reshape(1, 8 * C)

    MB2 = 2000
    Yblk = pl.pallas_call(
        _comb_body,
        grid=((_NB * V) // MB2,),
        in_specs=[
            pl.BlockSpec((MB2, 8 * C), lambda m: (m, 0)),
            pl.BlockSpec((2, MB2, 8 * C), lambda m: (0, m, 0)),
            pl.BlockSpec((8 * C, 8 * C), lambda m: (0, 0)),
            pl.BlockSpec((8 * C, 8 * C), lambda m: (0, 0)),
            pl.BlockSpec((1, 8 * C), lambda m: (0, 0)),
            pl.BlockSpec((1, 8 * C), lambda m: (0, 0)),
        ],
        out_specs=pl.BlockSpec((MB2, 8 * C), lambda m: (m, 0)),
        out_shape=jax.ShapeDtypeStruct((_NB * V, 8 * C), jnp.float32),
    )(X5, agg, BDroot, BDconv, biasb, convbb)

    out = Yblk.reshape(V, T, N, C)
    return jnp.transpose(out, (2, 0, 3, 1))


# speedup vs baseline: 1.1889x; 1.1889x over previous
import functools
import jax
import jax.numpy as jnp
from jax import lax
from jax.experimental import pallas as pl
from jax.experimental.pallas import tpu as pltpu
from jax.experimental.pallas import tpu_sc as plsc

_KD = 5
_NB = 4
_NQ = 8


def _elu(v):
    return jnp.where(v > 0, v, jnp.exp(v) - 1.0)


def _bd8(m):
    eye = jnp.eye(8, dtype=jnp.float32)
    return (eye[:, None, :, None] * m.astype(jnp.float32)[None, :, None, :]).reshape(128, 128)


def _prep_body(E, V, c_ref, a0_ref, a1_ref, gb_ref, wg_ref):
    c = c_ref[...]
    a0 = a0_ref[...]
    a1 = a1_ref[...]
    v0 = a0 * (_KD - 1.0)
    b0 = jnp.floor(v0)
    f0 = v0 - b0
    i0 = b0.astype(jnp.int32)
    v1 = a1 * (_KD - 1.0)
    b1 = jnp.floor(v1)
    f1 = v1 - b1
    i1 = b1.astype(jnp.int32)
    valid = lax.broadcasted_iota(jnp.int32, c.shape, 1) < E
    for s in range(4):
        bit0 = s & 1
        bit1 = (s >> 1) & 1
        basis = (f0 if bit0 else 1.0 - f0) * (f1 if bit1 else 1.0 - f1)
        wi = jnp.clip(i0 + bit0, 0, _KD - 1) + _KD * jnp.clip(i1 + bit1, 0, _KD - 1)
        gb_ref[s : s + 1, :] = (wi * V + c) * _NB
        wg_ref[s : s + 1, :] = jnp.where(valid, basis, 0.0)


def _y_body(x_ref, w_ref, y_ref):
    y_ref[...] = jnp.dot(x_ref[...], w_ref[0], preferred_element_type=jnp.float32)


def _comb_body(x_ref, agg_ref, root_ref, cw_ref, b_ref, cb_ref, o_ref):
    xb = x_ref[...]
    agg = agg_ref[0] + agg_ref[1]
    h = agg + jnp.dot(xb, root_ref[...], preferred_element_type=jnp.float32) + b_ref[...]
    h = _elu(h)
    res = _elu(jnp.dot(xb, cw_ref[...], preferred_element_type=jnp.float32) + cb_ref[...])
    o_ref[...] = _elu(h + res)


def _make_sc_agg(V, C, EP):
    ESL = EP // _NQ
    SCH = 256
    EB = 16
    RB = 4 * EB
    BPS = SCH // EB
    NSC = ESL // SCH
    GRP = 128
    NGRP = ESL // GRP
    ROWS_SH = _NB * V
    mesh = plsc.VectorSubcoreMesh(core_axis_name="c", subcore_axis_name="s")
    NC, NS = mesh.num_cores, mesh.num_subcores
    ZT = 10
    ZR = ROWS_SH // ZT

    @functools.partial(
        pl.kernel,
        out_type=jax.ShapeDtypeStruct((NC, ROWS_SH, 8 * C), jnp.float32),
        mesh=mesh,
        scratch_types=[
            pltpu.VMEM_SHARED((ROWS_SH, 8 * C), jnp.float32),
            pltpu.VMEM((8, 128), jnp.int32),
            pltpu.VMEM((8, 128), jnp.int32),
            pltpu.VMEM((NGRP, 128), jnp.int32),
            pltpu.VMEM((4 * SCH + 16,), jnp.float32),
            pltpu.VMEM((4 * SCH + 16,), jnp.float32),
            pltpu.VMEM((2, RB, 8 * C), jnp.float32),
            pltpu.VMEM((GRP, 8 * C), jnp.float32),
            pltpu.SemaphoreType.DMA,
            pltpu.SemaphoreType.DMA,
            pltpu.SemaphoreType.DMA,
        ],
    )
    def sc_agg(y5, gb2, rme2, wg, out_hbm, acc_sh, gbc, gbn, rmv, wgc, wgn,
               rows, cbuf, sem_g, sem_s, sem_m):
        cid = lax.axis_index("c")
        sid = lax.axis_index("s")
        b = sid % _NB
        q = cid * (_NQ // NC) + sid // _NB
        bv = b * V

        zero16 = jnp.zeros((16,), jnp.float32)

        def zb(i, carry):
            for j8 in range(8):
                cbuf[i, pl.ds(j8 * 16, 16)] = zero16
            return carry

        lax.fori_loop(0, GRP, zb, 0)

        @pl.when(sid < ZT)
        def _():
            z0 = sid * ZR
            done = 0
            while done < ZR:
                n = min(GRP, ZR - done)
                pltpu.sync_copy(cbuf.at[pl.ds(0, n)], acc_sh.at[pl.ds(z0 + done, n)])
                done += n

        plsc.subcore_barrier()

        prow = pl.multiple_of(q * (4 * ESL // 128), 8)
        ppair = pl.multiple_of(q * 4 * ESL, 128)
        prme = pl.multiple_of(q * NGRP, 8)

        pltpu.sync_copy(rme2.at[pl.ds(prme, NGRP)], rmv)
        for j in range(NGRP):
            for l in range(8):
                sl = pl.ds(l * 16, 16)
                rmv[j, sl] = rmv[j, sl] + bv

        def issue_meta(s):
            ro = pl.multiple_of(prow + s * 8, 8)
            pltpu.async_copy(gb2.at[pl.ds(ro, 8)], gbn, sem_m)
            pltpu.async_copy(wg.at[pl.ds(ppair + s * 4 * SCH, 4 * SCH)], wgn.at[pl.ds(0, 4 * SCH)], sem_m)

        def drain_meta():
            pltpu.make_async_copy(gb2.at[pl.ds(0, 8)], gbn, sem_m).wait()
            pltpu.make_async_copy(wg.at[pl.ds(0, 4 * SCH)], wgn.at[pl.ds(0, 4 * SCH)], sem_m).wait()

        def adopt_meta():
            for j in range(8):
                for l in range(8):
                    sl = pl.ds(l * 16, 16)
                    gbc[j, sl] = gbn[j, sl] + b
            def wcp(i, carry):
                sl = pl.ds(i * 16, 16)
                wgc[sl] = wgn[sl]
                return carry
            lax.fori_loop(0, (4 * SCH) // 16, wcp, 0)

        def issue_gather(u):
            idx = gbc.at[u // 2, pl.ds((u % 2) * RB, RB)]
            return pltpu.async_copy(y5.at[idx], rows.at[u % 2], sem_g)

        def drain_scatter():
            pltpu.make_async_copy(out_hbm.at[0, pl.ds(0, GRP)], cbuf, sem_s).wait()

        def combine(u):
            cb0 = (u % 8) * EB
            p = u % 2
            woff = u * 4 * EB
            for half in range(2):
                hw = woff + half * 32
                wvecs = [wgc[pl.ds(hw + i * 16, 16)] for i in range(2)]
                ws = [v[i] for v in wvecs for i in range(16)]
                hr = half * 32
                hc = cb0 + half * 8

                def jb(j, carry):
                    sl = pl.ds(j * 16, 16)
                    for e in range(8):
                        rb = hr + e * 4
                        v = (rows[p, rb, sl] * ws[4 * e] + rows[p, rb + 1, sl] * ws[4 * e + 1]
                             + rows[p, rb + 2, sl] * ws[4 * e + 2] + rows[p, rb + 3, sl] * ws[4 * e + 3])
                        cbuf[hc + e, sl] = v
                    return carry

                lax.fori_loop(0, 8, jb, 0)

        pltpu.sync_copy(gb2.at[pl.ds(prow, 8)], gbn)
        pltpu.sync_copy(wg.at[pl.ds(ppair, 4 * SCH)], wgn.at[pl.ds(0, 4 * SCH)])
        adopt_meta()
        issue_meta(1)

        def body(s, carry):
            @pl.when(s > 0)
            def _():
                drain_scatter()

            pend = [issue_gather(0)]
            for u in range(BPS):
                if u + 1 < BPS:
                    pend.append(issue_gather(u + 1))
                pend[u].wait()
                if u == BPS // 2:
                    drain_scatter()
                combine(u)
                if u == BPS // 2 - 1:
                    pltpu.async_copy(cbuf, acc_sh.at[rmv.at[2 * s]], sem_s, add=True)
                if u == BPS - 1:
                    pltpu.async_copy(cbuf, acc_sh.at[rmv.at[2 * s + 1]], sem_s, add=True)

            @pl.when(s + 1 < NSC)
            def _():
                drain_meta()
                adopt_meta()

                @pl.when(s + 2 < NSC)
                def _():
                    issue_meta(s + 2)

            return carry

        lax.fori_loop(0, NSC, body, 0)
        drain_scatter()
        plsc.subcore_barrier()

        @pl.when(sid < ZT)
        def _():
            pltpu.sync_copy(
                acc_sh.at[pl.ds(sid * ZR, ZR)],
                out_hbm.at[cid, pl.ds(sid * ZR, ZR)],
            )

    return sc_agg


def kernel(x, edge_index, edge_attr, W, root, bias, conv_w, conv_b):
    N, V, C, T = x.shape
    G = N * T
    E = edge_index.shape[1] // N
    K = W.shape[0]
    EP = ((E + 2047) // 2048) * 2048
    PAIRS = 4 * EP

    x = x.astype(jnp.float32)
    xt = jnp.transpose(x, (1, 3, 0, 2)).reshape(V, G, C)
    X5 = xt.reshape(V * _NB, 8 * C)
    Xblk = jnp.transpose(xt.reshape(V, _NB, 8 * C), (1, 0, 2)).reshape(_NB * V, 8 * C)

    r = edge_index[0, :E].astype(jnp.int32)
    c = edge_index[1, :E].astype(jnp.int32)
    cp_ = jnp.pad(c, (0, EP - E)).reshape(1, EP)
    a0 = jnp.pad(edge_attr[:E, 0].astype(jnp.float32), (0, EP - E)).reshape(1, EP)
    a1 = jnp.pad(edge_attr[:E, 1].astype(jnp.float32), (0, EP - E)).reshape(1, EP)

    gb, wg = pl.pallas_call(
        functools.partial(_prep_body, E, V),
        out_shape=[
            jax.ShapeDtypeStruct((4, EP), jnp.int32),
            jax.ShapeDtypeStruct((4, EP), jnp.float32),
        ],
    )(cp_, a0, a1)

    gb2 = jnp.transpose(gb).reshape(PAIRS // 128, 128)
    wgf = jnp.transpose(wg).reshape(PAIRS)
    rme2 = jnp.pad(r, (0, EP - E)).reshape(EP // 128, 128)

    eye8 = jnp.eye(8, dtype=jnp.float32)
    BDW = (eye8[None, :, None, :, None]
           * W.astype(jnp.float32)[:, None, :, None, :]).reshape(K, 8 * C, 8 * C)
    Y5 = pl.pallas_call(
        _y_body,
        grid=(K,),
        in_specs=[
            pl.BlockSpec((V * _NB, 8 * C), lambda k: (0, 0)),
            pl.BlockSpec((1, 8 * C, 8 * C), lambda k: (k, 0, 0)),
        ],
        out_specs=pl.BlockSpec((V * _NB, 8 * C), lambda k: (k, 0)),
        out_shape=jax.ShapeDtypeStruct((K * V * _NB, 8 * C), jnp.float32),
    )(X5, BDW)

    agg = _make_sc_agg(V, C, EP)(Y5, gb2, rme2, wgf)

    BDroot = _bd8(root)
    BDconv = _bd8(jnp.transpose(conv_w))
    biasb = jnp.tile(bias.astype(jnp.float32), 8).reshape(1, 8 * C)
    convbb = jnp.tile(conv_b.astype(jnp.float32), 8).reshape(1, 8 * C)

    MB2 = 2000
    Yblk = pl.pallas_call(
        _comb_body,
        grid=((_NB * V) // MB2,),
        in_specs=[
            pl.BlockSpec((MB2, 8 * C), lambda m: (m, 0)),
            pl.BlockSpec((2, MB2, 8 * C), lambda m: (0, m, 0)),
            pl.BlockSpec((8 * C, 8 * C), lambda m: (0, 0)),
            pl.BlockSpec((8 * C, 8 * C), lambda m: (0, 0)),
            pl.BlockSpec((1, 8 * C), lambda m: (0, 0)),
            pl.BlockSpec((1, 8 * C), lambda m: (0, 0)),
        ],
        out_specs=pl.BlockSpec((MB2, 8 * C), lambda m: (m, 0)),
        out_shape=jax.ShapeDtypeStruct((_NB * V, 8 * C), jnp.float32),
    )(Xblk, agg, BDroot, BDconv, biasb, convbb)

    out = Yblk.reshape(_NB, V, 8, C)
    out = jnp.transpose(out, (1, 0, 2, 3)).reshape(V, T, N, C)
    return jnp.transpose(out, (2, 0, 3, 1))


# speedup vs baseline: 1.2627x; 1.0621x over previous
import functools
import jax
import jax.numpy as jnp
from jax import lax
from jax.experimental import pallas as pl
from jax.experimental.pallas import tpu as pltpu
from jax.experimental.pallas import tpu_sc as plsc

_KD = 5
_NB = 4
_NQ = 8


def _elu(v):
    return jnp.where(v > 0, v, jnp.exp(v) - 1.0)


def _bd8(m):
    eye = jnp.eye(8, dtype=jnp.float32)
    return (eye[:, None, :, None] * m.astype(jnp.float32)[None, :, None, :]).reshape(128, 128)


def _prep_body(E, V, c_ref, a0_ref, a1_ref, gb_ref, wg_ref):
    c = c_ref[...]
    a0 = a0_ref[...]
    a1 = a1_ref[...]
    v0 = a0 * (_KD - 1.0)
    b0 = jnp.floor(v0)
    f0 = v0 - b0
    i0 = b0.astype(jnp.int32)
    v1 = a1 * (_KD - 1.0)
    b1 = jnp.floor(v1)
    f1 = v1 - b1
    i1 = b1.astype(jnp.int32)
    valid = lax.broadcasted_iota(jnp.int32, c.shape, 1) < E
    for s in range(4):
        bit0 = s & 1
        bit1 = (s >> 1) & 1
        basis = (f0 if bit0 else 1.0 - f0) * (f1 if bit1 else 1.0 - f1)
        wi = jnp.clip(i0 + bit0, 0, _KD - 1) + _KD * jnp.clip(i1 + bit1, 0, _KD - 1)
        gb_ref[s : s + 1, :] = (wi * V + c) * _NB
        wg_ref[s : s + 1, :] = jnp.where(valid, basis, 0.0)


def _y_body(x_ref, w_ref, y_ref):
    y_ref[...] = jnp.dot(x_ref[...], w_ref[0], preferred_element_type=jnp.float32)


def _comb_body(x_ref, agg_ref, root_ref, cw_ref, b_ref, cb_ref, o_ref):
    xb = x_ref[...]
    agg = agg_ref[0] + agg_ref[1]
    h = agg + jnp.dot(xb, root_ref[...], preferred_element_type=jnp.float32) + b_ref[...]
    h = _elu(h)
    res = _elu(jnp.dot(xb, cw_ref[...], preferred_element_type=jnp.float32) + cb_ref[...])
    o_ref[...] = _elu(h + res)


def _make_sc_agg(V, C, EP):
    ESL = EP // _NQ
    SCH = 256
    EB = 16
    RB = 4 * EB
    BPS = SCH // EB
    NSC = ESL // SCH
    GRP = 128
    NGRP = ESL // GRP
    ROWS_SH = _NB * V
    mesh = plsc.VectorSubcoreMesh(core_axis_name="c", subcore_axis_name="s")
    NC, NS = mesh.num_cores, mesh.num_subcores
    ZT = 10
    ZR = ROWS_SH // ZT

    @functools.partial(
        pl.kernel,
        out_type=jax.ShapeDtypeStruct((NC, ROWS_SH, 8 * C), jnp.float32),
        mesh=mesh,
        scratch_types=[
            pltpu.VMEM_SHARED((ROWS_SH, 8 * C), jnp.float32),
            pltpu.VMEM((8, 128), jnp.int32),
            pltpu.VMEM((8, 128), jnp.int32),
            pltpu.VMEM((NGRP, 128), jnp.int32),
            pltpu.VMEM((4 * SCH + 16,), jnp.float32),
            pltpu.VMEM((4 * SCH + 16,), jnp.float32),
            pltpu.VMEM((3, RB, 8 * C), jnp.float32),
            pltpu.VMEM((GRP, 8 * C), jnp.float32),
            pltpu.SemaphoreType.DMA,
            pltpu.SemaphoreType.DMA,
            pltpu.SemaphoreType.DMA,
        ],
    )
    def sc_agg(y5, gb2, rme2, wg, out_hbm, acc_sh, gbc, gbn, rmv, wgc, wgn,
               rows, cbuf, sem_g, sem_s, sem_m):
        cid = lax.axis_index("c")
        sid = lax.axis_index("s")
        b = sid % _NB
        q = cid * (_NQ // NC) + sid // _NB
        bv = b * V

        zero16 = jnp.zeros((16,), jnp.float32)

        def zb(i, carry):
            for j8 in range(8):
                cbuf[i, pl.ds(j8 * 16, 16)] = zero16
            return carry

        lax.fori_loop(0, GRP, zb, 0)

        @pl.when(sid < ZT)
        def _():
            z0 = sid * ZR
            done = 0
            while done < ZR:
                n = min(GRP, ZR - done)
                pltpu.sync_copy(cbuf.at[pl.ds(0, n)], acc_sh.at[pl.ds(z0 + done, n)])
                done += n

        plsc.subcore_barrier()

        prow = pl.multiple_of(q * (4 * ESL // 128), 8)
        ppair = pl.multiple_of(q * 4 * ESL, 128)
        prme = pl.multiple_of(q * NGRP, 8)

        pltpu.sync_copy(rme2.at[pl.ds(prme, NGRP)], rmv)
        for j in range(NGRP):
            for l in range(8):
                sl = pl.ds(l * 16, 16)
                rmv[j, sl] = rmv[j, sl] + bv

        def issue_meta(s):
            ro = pl.multiple_of(prow + s * 8, 8)
            pltpu.async_copy(gb2.at[pl.ds(ro, 8)], gbn, sem_m)
            pltpu.async_copy(wg.at[pl.ds(ppair + s * 4 * SCH, 4 * SCH)], wgn.at[pl.ds(0, 4 * SCH)], sem_m)

        def drain_meta():
            pltpu.make_async_copy(gb2.at[pl.ds(0, 8)], gbn, sem_m).wait()
            pltpu.make_async_copy(wg.at[pl.ds(0, 4 * SCH)], wgn.at[pl.ds(0, 4 * SCH)], sem_m).wait()

        def adopt_meta():
            for j in range(8):
                for l in range(8):
                    sl = pl.ds(l * 16, 16)
                    gbc[j, sl] = gbn[j, sl] + b
            def wcp(i, carry):
                sl = pl.ds(i * 16, 16)
                wgc[sl] = wgn[sl]
                return carry
            lax.fori_loop(0, (4 * SCH) // 16, wcp, 0)

        def issue_gather(u):
            idx = gbc.at[u // 2, pl.ds((u % 2) * RB, RB)]
            return pltpu.async_copy(y5.at[idx], rows.at[u % 3], sem_g)

        def drain_scatter():
            pltpu.make_async_copy(out_hbm.at[0, pl.ds(0, GRP)], cbuf, sem_s).wait()

        def combine(u):
            cb0 = (u % 8) * EB
            p = u % 3
            woff = u * 4 * EB
            for half in range(2):
                hw = woff + half * 32
                wvecs = [wgc[pl.ds(hw + i * 16, 16)] for i in range(2)]
                ws = [v[i] for v in wvecs for i in range(16)]
                hr = half * 32
                hc = cb0 + half * 8

                def jb(j, carry):
                    sl = pl.ds(j * 16, 16)
                    for e in range(8):
                        rb = hr + e * 4
                        v = (rows[p, rb, sl] * ws[4 * e] + rows[p, rb + 1, sl] * ws[4 * e + 1]
                             + rows[p, rb + 2, sl] * ws[4 * e + 2] + rows[p, rb + 3, sl] * ws[4 * e + 3])
                        cbuf[hc + e, sl] = v
                    return carry

                lax.fori_loop(0, 8, jb, 0)

        pltpu.sync_copy(gb2.at[pl.ds(prow, 8)], gbn)
        pltpu.sync_copy(wg.at[pl.ds(ppair, 4 * SCH)], wgn.at[pl.ds(0, 4 * SCH)])
        adopt_meta()
        issue_meta(1)

        def body(s, carry):
            @pl.when(s > 0)
            def _():
                drain_scatter()

            pend = [issue_gather(0), issue_gather(1)]
            for u in range(BPS):
                if u + 2 < BPS:
                    pend.append(issue_gather(u + 2))
                pend[u].wait()
                if u == BPS // 2:
                    drain_scatter()
                combine(u)
                if u == BPS // 2 - 1:
                    pltpu.async_copy(cbuf, acc_sh.at[rmv.at[2 * s]], sem_s, add=True)
                if u == BPS - 1:
                    pltpu.async_copy(cbuf, acc_sh.at[rmv.at[2 * s + 1]], sem_s, add=True)

            @pl.when(s + 1 < NSC)
            def _():
                drain_meta()
                adopt_meta()

                @pl.when(s + 2 < NSC)
                def _():
                    issue_meta(s + 2)

            return carry

        lax.fori_loop(0, NSC, body, 0)
        drain_scatter()
        plsc.subcore_barrier()

        @pl.when(sid < ZT)
        def _():
            pltpu.sync_copy(
                acc_sh.at[pl.ds(sid * ZR, ZR)],
                out_hbm.at[cid, pl.ds(sid * ZR, ZR)],
            )

    return sc_agg


def kernel(x, edge_index, edge_attr, W, root, bias, conv_w, conv_b):
    N, V, C, T = x.shape
    G = N * T
    E = edge_index.shape[1] // N
    K = W.shape[0]
    EP = ((E + 2047) // 2048) * 2048
    PAIRS = 4 * EP

    x = x.astype(jnp.float32)
    xt = jnp.transpose(x, (1, 3, 0, 2)).reshape(V, G, C)
    X5 = xt.reshape(V * _NB, 8 * C)
    Xblk = jnp.transpose(xt.reshape(V, _NB, 8 * C), (1, 0, 2)).reshape(_NB * V, 8 * C)

    r = edge_index[0, :E].astype(jnp.int32)
    c = edge_index[1, :E].astype(jnp.int32)
    cp_ = jnp.pad(c, (0, EP - E)).reshape(1, EP)
    a0 = jnp.pad(edge_attr[:E, 0].astype(jnp.float32), (0, EP - E)).reshape(1, EP)
    a1 = jnp.pad(edge_attr[:E, 1].astype(jnp.float32), (0, EP - E)).reshape(1, EP)

    gb, wg = pl.pallas_call(
        functools.partial(_prep_body, E, V),
        out_shape=[
            jax.ShapeDtypeStruct((4, EP), jnp.int32),
            jax.ShapeDtypeStruct((4, EP), jnp.float32),
        ],
    )(cp_, a0, a1)

    gb2 = jnp.transpose(gb).reshape(PAIRS // 128, 128)
    wgf = jnp.transpose(wg).reshape(PAIRS)
    rme2 = jnp.pad(r, (0, EP - E)).reshape(EP // 128, 128)

    eye8 = jnp.eye(8, dtype=jnp.float32)
    BDW = (eye8[None, :, None, :, None]
           * W.astype(jnp.float32)[:, None, :, None, :]).reshape(K, 8 * C, 8 * C)
    Y5 = pl.pallas_call(
        _y_body,
        grid=(K,),
        in_specs=[
            pl.BlockSpec((V * _NB, 8 * C), lambda k: (0, 0)),
            pl.BlockSpec((1, 8 * C, 8 * C), lambda k: (k, 0, 0)),
        ],
        out_specs=pl.BlockSpec((V * _NB, 8 * C), lambda k: (k, 0)),
        out_shape=jax.ShapeDtypeStruct((K * V * _NB, 8 * C), jnp.float32),
    )(X5, BDW)

    agg = _make_sc_agg(V, C, EP)(Y5, gb2, rme2, wgf)

    BDroot = _bd8(root)
    BDconv = _bd8(jnp.transpose(conv_w))
    biasb = jnp.tile(bias.astype(jnp.float32), 8).reshape(1, 8 * C)
    convbb = jnp.tile(conv_b.astype(jnp.float32), 8).reshape(1, 8 * C)

    MB2 = 2000
    Yblk = pl.pallas_call(
        _comb_body,
        grid=((_NB * V) // MB2,),
        in_specs=[
            pl.BlockSpec((MB2, 8 * C), lambda m: (m, 0)),
            pl.BlockSpec((2, MB2, 8 * C), lambda m: (0, m, 0)),
            pl.BlockSpec((8 * C, 8 * C), lambda m: (0, 0)),
            pl.BlockSpec((8 * C, 8 * C), lambda m: (0, 0)),
            pl.BlockSpec((1, 8 * C), lambda m: (0, 0)),
            pl.BlockSpec((1, 8 * C), lambda m: (0, 0)),
        ],
        out_specs=pl.BlockSpec((MB2, 8 * C), lambda m: (m, 0)),
        out_shape=jax.ShapeDtypeStruct((_NB * V, 8 * C), jnp.float32),
    )(Xblk, agg, BDroot, BDconv, biasb, convbb)

    out = Yblk.reshape(_NB, V, 8, C)
    out = jnp.transpose(out, (1, 0, 2, 3)).reshape(V, T, N, C)
    return jnp.transpose(out, (2, 0, 3, 1))
